# baseline (speedup 1.0000x reference)
import functools
import jax
import jax.numpy as jnp
from jax import lax
from jax.experimental import pallas as pl
from jax.experimental.pallas import tpu as pltpu
from jax.experimental.pallas import tpu_sc as plsc

N_NODES = 10000
N_EDGES = 320000
D = 128
D_OUT = 32
EC = 128
NW = 32
CPW = 80
NP = 10240
RPT = NP // 16
ZR = 64

_mesh = plsc.VectorSubcoreMesh(core_axis_name="c", subcore_axis_name="s",
                               num_cores=2)


@functools.partial(
    pl.kernel,
    mesh=_mesh,
    out_type=jax.ShapeDtypeStruct((2, NP, D), jnp.float32),
    scratch_types=[
        pltpu.VMEM((CPW, EC), jnp.int32),
        pltpu.VMEM((EC, D), jnp.float32),
        pltpu.VMEM_SHARED((NP, D), jnp.float32),
        pltpu.SemaphoreType.DMA,
    ],
)
def _sc_deg(dst_hbm, deg_out, dst_v, ones_v, deg_sh, sem):
    cid = lax.axis_index("c")
    sid = lax.axis_index("s")
    wid = cid * 16 + sid

    def zrow(r, _):
        for j in range(8):
            ones_v[r, pl.ds(j * 16, 16)] = jnp.zeros((16,), jnp.float32)
        return 0
    lax.fori_loop(0, ZR, zrow, 0)

    for k in range(RPT // ZR):
        off = pl.multiple_of(sid * RPT + k * ZR, ZR)
        pltpu.sync_copy(ones_v.at[pl.ds(0, ZR)], deg_sh.at[pl.ds(off, ZR)])

    def orow(r, _):
        for j in range(8):
            ones_v[r, pl.ds(j * 16, 16)] = jnp.full((16,), 1.0, jnp.float32)
        return 0
    lax.fori_loop(0, EC, orow, 0)

    c0 = pl.multiple_of(wid * CPW, 8)
    pltpu.sync_copy(dst_hbm.at[pl.ds(c0, CPW)], dst_v)

    plsc.subcore_barrier()

    def body(j, _):
        pltpu.async_copy(ones_v, deg_sh.at[dst_v.at[j]], sem, add=True)
        return 0
    lax.fori_loop(0, CPW, body, 0)

    def drain(j, _):
        pltpu.make_async_copy(ones_v, deg_sh.at[dst_v.at[j]], sem).wait()
        return 0
    lax.fori_loop(0, CPW, drain, 0)

    plsc.subcore_barrier()

    r0 = pl.multiple_of(sid * RPT, ZR)
    pltpu.sync_copy(deg_sh.at[pl.ds(r0, RPT)],
                    deg_out.at[cid, pl.ds(r0, RPT)])


@functools.partial(
    pl.kernel,
    mesh=_mesh,
    out_type=jax.ShapeDtypeStruct((2, NP, D), jnp.float32),
    scratch_types=[
        pltpu.VMEM((CPW // 2, EC), jnp.int32),
        pltpu.VMEM((CPW // 2, EC), jnp.int32),
        pltpu.VMEM((EC, D), jnp.float32),
        pltpu.VMEM((EC, D), jnp.float32),
        pltpu.VMEM_SHARED((NP, D), jnp.float32),
        pltpu.SemaphoreType.DMA,
        pltpu.SemaphoreType.DMA,
        pltpu.SemaphoreType.DMA,
        pltpu.SemaphoreType.DMA,
    ],
)
def _sc_agg(h_hbm, src_hbm, dst_hbm, agg_out,
            src_v, dst_v, rows_a, rows_b, agg_sh,
            gsa, gsb, ssa, ssb):
    cid = lax.axis_index("c")
    sid = lax.axis_index("s")
    wid = cid * 16 + sid
    hp = CPW // 2

    def zrow(r, _):
        for j in range(8):
            rows_a[r, pl.ds(j * 16, 16)] = jnp.zeros((16,), jnp.float32)
        return 0
    lax.fori_loop(0, EC, zrow, 0)

    for k in range(RPT // EC):
        off = pl.multiple_of(sid * RPT + k * EC, ZR)
        pltpu.sync_copy(rows_a, agg_sh.at[pl.ds(off, EC)])

    plsc.subcore_barrier()

    def wait_gather(rows, sem2, j):
        pltpu.make_async_copy(h_hbm.at[src_v.at[j]], rows, sem2).wait()

    def wait_scatter(rows, sem2, j):
        pltpu.make_async_copy(rows, agg_sh.at[dst_v.at[j]], sem2).wait()

    for phase in range(2):
        c0 = pl.multiple_of(wid * CPW + phase * hp, 8)
        pltpu.sync_copy(src_hbm.at[pl.ds(c0, hp)], src_v)
        pltpu.sync_copy(dst_hbm.at[pl.ds(c0, hp)], dst_v)

        pltpu.async_copy(h_hbm.at[src_v.at[0]], rows_a, gsa)
        pltpu.async_copy(h_hbm.at[src_v.at[1]], rows_b, gsb)

        def body(g, _):
            j = 2 * g
            wait_gather(rows_a, gsa, j)
            pltpu.async_copy(rows_a, agg_sh.at[dst_v.at[j]], ssa, add=True)
            wait_gather(rows_b, gsb, j + 1)
            pltpu.async_copy(rows_b, agg_sh.at[dst_v.at[j + 1]], ssb,
                             add=True)
            wait_scatter(rows_a, ssa, j)
            pltpu.async_copy(h_hbm.at[src_v.at[j + 2]], rows_a, gsa)
            wait_scatter(rows_b, ssb, j + 1)
            pltpu.async_copy(h_hbm.at[src_v.at[j + 3]], rows_b, gsb)
            return 0

        lax.fori_loop(0, hp // 2 - 1, body, 0)

        j = hp - 2
        wait_gather(rows_a, gsa, j)
        pltpu.async_copy(rows_a, agg_sh.at[dst_v.at[j]], ssa, add=True)
        wait_gather(rows_b, gsb, j + 1)
        pltpu.async_copy(rows_b, agg_sh.at[dst_v.at[j + 1]], ssb, add=True)
        wait_scatter(rows_a, ssa, j)
        wait_scatter(rows_b, ssb, j + 1)

    plsc.subcore_barrier()

    r0 = pl.multiple_of(sid * RPT, ZR)
    pltpu.sync_copy(agg_sh.at[pl.ds(r0, RPT)],
                    agg_out.at[cid, pl.ds(r0, RPT)])


BM = 400


def _tc_scale_kernel(deg_ref, o_ref):
    d = deg_ref[0, :, 0:1] + deg_ref[1, :, 0:1]
    o_ref[...] = 1.0 / (d + 1.0) * jnp.ones((1, D), jnp.float32)


def _tc_scale(deg):
    return pl.pallas_call(
        _tc_scale_kernel,
        grid=(N_NODES // BM,),
        in_specs=[pl.BlockSpec((2, BM, D), lambda i: (0, i, 0))],
        out_specs=pl.BlockSpec((BM, D), lambda i: (i, 0)),
        out_shape=jax.ShapeDtypeStruct((N_NODES, D), jnp.float32),
    )(deg)


def _tc_layer_kernel(agg_ref, h_ref, s_ref, w_ref, b_ref, o_ref):
    z = (agg_ref[0] + agg_ref[1] + h_ref[...]) * s_ref[...]
    o_ref[...] = jnp.maximum(
        jnp.dot(z, w_ref[...], preferred_element_type=jnp.float32)
        + b_ref[...], 0.0)


def _tc_layer(agg, h, scale, W, b):
    return pl.pallas_call(
        _tc_layer_kernel,
        grid=(N_NODES // BM,),
        in_specs=[
            pl.BlockSpec((2, BM, D), lambda i: (0, i, 0)),
            pl.BlockSpec((BM, D), lambda i: (i, 0)),
            pl.BlockSpec((BM, D), lambda i: (i, 0)),
            pl.BlockSpec((D, D), lambda i: (0, 0)),
            pl.BlockSpec((1, D), lambda i: (0, 0)),
        ],
        out_specs=pl.BlockSpec((BM, D), lambda i: (i, 0)),
        out_shape=jax.ShapeDtypeStruct((N_NODES, D), jnp.float32),
    )(agg, h, scale, W, b.reshape(1, D))


def _tc_final_kernel(agg_ref, h_ref, s_ref, w3_ref, b3_ref, wv_ref,
                     bv_ref, wc_ref, bc_ref, o_ref, acc):
    i = pl.program_id(0)
    z = (agg_ref[0] + agg_ref[1] + h_ref[...]) * s_ref[...]
    h3 = jnp.dot(z, w3_ref[...], preferred_element_type=jnp.float32) \
        + b3_ref[...]
    wv = jax.nn.sigmoid(
        jnp.dot(h3, wv_ref[...], preferred_element_type=jnp.float32)
        + bv_ref[...])
    part = jnp.sum(wv * h3, axis=0, keepdims=True)

    @pl.when(i == 0)
    def _():
        acc[...] = jnp.zeros_like(acc)

    acc[...] += part

    @pl.when(i == pl.num_programs(0) - 1)
    def _():
        hg = acc[...] / float(N_NODES)
        o_ref[...] = jnp.dot(hg, wc_ref[...],
                             preferred_element_type=jnp.float32) + bc_ref[...]


def _tc_final(agg, h, scale, W3, b3, Wv, bv, Wc, bc):
    return pl.pallas_call(
        _tc_final_kernel,
        grid=(N_NODES // BM,),
        in_specs=[
            pl.BlockSpec((2, BM, D), lambda i: (0, i, 0)),
            pl.BlockSpec((BM, D), lambda i: (i, 0)),
            pl.BlockSpec((BM, D), lambda i: (i, 0)),
            pl.BlockSpec((D, D), lambda i: (0, 0)),
            pl.BlockSpec((1, D), lambda i: (0, 0)),
            pl.BlockSpec((D, 1), lambda i: (0, 0)),
            pl.BlockSpec((1, 1), lambda i: (0, 0)),
            pl.BlockSpec((D, D_OUT), lambda i: (0, 0)),
            pl.BlockSpec((1, D_OUT), lambda i: (0, 0)),
        ],
        out_specs=pl.BlockSpec((1, D_OUT), lambda i: (0, 0)),
        out_shape=jax.ShapeDtypeStruct((1, D_OUT), jnp.float32),
        scratch_shapes=[pltpu.VMEM((1, D), jnp.float32)],
    )(agg, h, scale, W3, b3.reshape(1, D), Wv, bv.reshape(1, 1),
      Wc, bc.reshape(1, D_OUT))


@jax.jit
def _run(x, edge_index, W1, b1, W2, b2, W3, b3, Wv, bv, Wc, bc):
    npad = NW * CPW * EC - N_EDGES
    src = jnp.concatenate(
        [edge_index[0].astype(jnp.int32), jnp.zeros((npad,), jnp.int32)])
    dst = jnp.concatenate(
        [edge_index[1].astype(jnp.int32),
         N_NODES + (jnp.arange(npad, dtype=jnp.int32) % (NP - N_NODES))])
    src = src.reshape(NW * CPW, EC)
    dst = dst.reshape(NW * CPW, EC)

    deg = _sc_deg(dst)
    scale = _tc_scale(deg)
    agg1 = _sc_agg(x, src, dst)
    h1 = _tc_layer(agg1, x, scale, W1, b1)
    agg2 = _sc_agg(h1, src, dst)
    h2 = _tc_layer(agg2, h1, scale, W2, b2)
    agg3 = _sc_agg(h2, src, dst)
    return _tc_final(agg3, h2, scale, W3, b3, Wv, bv, Wc, bc)


def kernel(x, edge_index, W1, b1, W2, b2, W3, b3, Wv, bv, Wc, bc):
    return _run(x, edge_index, W1, b1, W2, b2, W3, b3, Wv, bv, Wc, bc)


# speedup vs baseline: 1.0845x; 1.0845x over previous
import functools
import jax
import jax.numpy as jnp
from jax import lax
from jax.experimental import pallas as pl
from jax.experimental.pallas import tpu as pltpu
from jax.experimental.pallas import tpu_sc as plsc

N_NODES = 10000
N_EDGES = 320000
D = 128
D_OUT = 32
EC = 128
NW = 32
CPW = 80
NP = 10240
RPT = NP // 16
ZR = 64

_mesh = plsc.VectorSubcoreMesh(core_axis_name="c", subcore_axis_name="s",
                               num_cores=2)


@functools.partial(
    pl.kernel,
    mesh=_mesh,
    out_type=jax.ShapeDtypeStruct((2, NP, D), jnp.float32),
    scratch_types=[
        pltpu.VMEM((CPW, EC), jnp.int32),
        pltpu.VMEM((EC, D), jnp.float32),
        pltpu.VMEM_SHARED((NP, D), jnp.float32),
        pltpu.SemaphoreType.DMA,
    ],
)
def _sc_deg(dst_hbm, deg_out, dst_v, ones_v, deg_sh, sem):
    cid = lax.axis_index("c")
    sid = lax.axis_index("s")
    wid = cid * 16 + sid

    def zrow(r, _):
        for j in range(8):
            ones_v[r, pl.ds(j * 16, 16)] = jnp.zeros((16,), jnp.float32)
        return 0
    lax.fori_loop(0, ZR, zrow, 0)

    for k in range(RPT // ZR):
        off = pl.multiple_of(sid * RPT + k * ZR, ZR)
        pltpu.sync_copy(ones_v.at[pl.ds(0, ZR)], deg_sh.at[pl.ds(off, ZR)])

    def orow(r, _):
        for j in range(8):
            ones_v[r, pl.ds(j * 16, 16)] = jnp.full((16,), 1.0, jnp.float32)
        return 0
    lax.fori_loop(0, EC, orow, 0)

    c0 = pl.multiple_of(wid * CPW, 8)
    pltpu.sync_copy(dst_hbm.at[pl.ds(c0, CPW)], dst_v)

    plsc.subcore_barrier()

    def body(j, _):
        pltpu.async_copy(ones_v, deg_sh.at[dst_v.at[j]], sem, add=True)
        return 0
    lax.fori_loop(0, CPW, body, 0)

    def drain(j, _):
        pltpu.make_async_copy(ones_v, deg_sh.at[dst_v.at[j]], sem).wait()
        return 0
    lax.fori_loop(0, CPW, drain, 0)

    plsc.subcore_barrier()

    r0 = pl.multiple_of(sid * RPT, ZR)
    pltpu.sync_copy(deg_sh.at[pl.ds(r0, RPT)],
                    deg_out.at[cid, pl.ds(r0, RPT)])


@functools.partial(
    pl.kernel,
    mesh=_mesh,
    out_type=jax.ShapeDtypeStruct((2, NP, D), jnp.float32),
    scratch_types=[
        pltpu.VMEM((CPW // 2, EC), jnp.int32),
        pltpu.VMEM((CPW // 2, EC), jnp.int32),
        pltpu.VMEM((EC, D), jnp.float32),
        pltpu.VMEM((EC, D), jnp.float32),
        pltpu.VMEM_SHARED((NP, D), jnp.float32),
        pltpu.SemaphoreType.DMA,
        pltpu.SemaphoreType.DMA,
        pltpu.SemaphoreType.DMA,
        pltpu.SemaphoreType.DMA,
    ],
)
def _sc_agg(h_hbm, src_hbm, dst_hbm, agg_out,
            src_v, dst_v, rows_a, rows_b, agg_sh,
            gsa, gsb, ssa, ssb):
    cid = lax.axis_index("c")
    sid = lax.axis_index("s")
    wid = cid * 16 + sid
    hp = CPW // 2

    def zrow(r, _):
        for j in range(8):
            rows_a[r, pl.ds(j * 16, 16)] = jnp.zeros((16,), jnp.float32)
        return 0
    lax.fori_loop(0, EC, zrow, 0)

    for k in range(RPT // EC):
        off = pl.multiple_of(sid * RPT + k * EC, ZR)
        pltpu.sync_copy(rows_a, agg_sh.at[pl.ds(off, EC)])

    plsc.subcore_barrier()

    def wait_gather(rows, sem2, j):
        pltpu.make_async_copy(h_hbm.at[src_v.at[j]], rows, sem2).wait()

    def wait_scatter(rows, sem2, j):
        pltpu.make_async_copy(rows, agg_sh.at[dst_v.at[j]], sem2).wait()

    for phase in range(2):
        c0 = pl.multiple_of(wid * CPW + phase * hp, 8)
        pltpu.sync_copy(src_hbm.at[pl.ds(c0, hp)], src_v)
        pltpu.sync_copy(dst_hbm.at[pl.ds(c0, hp)], dst_v)

        pltpu.async_copy(h_hbm.at[src_v.at[0]], rows_a, gsa)
        pltpu.async_copy(h_hbm.at[src_v.at[1]], rows_b, gsb)

        def body(g, _):
            j = 2 * g
            wait_gather(rows_a, gsa, j)
            pltpu.async_copy(rows_a, agg_sh.at[dst_v.at[j]], ssa, add=True)
            wait_gather(rows_b, gsb, j + 1)
            pltpu.async_copy(rows_b, agg_sh.at[dst_v.at[j + 1]], ssb,
                             add=True)
            wait_scatter(rows_a, ssa, j)
            pltpu.async_copy(h_hbm.at[src_v.at[j + 2]], rows_a, gsa)
            wait_scatter(rows_b, ssb, j + 1)
            pltpu.async_copy(h_hbm.at[src_v.at[j + 3]], rows_b, gsb)
            return 0

        lax.fori_loop(0, hp // 2 - 1, body, 0)

        j = hp - 2
        wait_gather(rows_a, gsa, j)
        pltpu.async_copy(rows_a, agg_sh.at[dst_v.at[j]], ssa, add=True)
        wait_gather(rows_b, gsb, j + 1)
        pltpu.async_copy(rows_b, agg_sh.at[dst_v.at[j + 1]], ssb, add=True)
        wait_scatter(rows_a, ssa, j)
        wait_scatter(rows_b, ssb, j + 1)

    plsc.subcore_barrier()

    r0 = pl.multiple_of(sid * RPT, ZR)
    pltpu.sync_copy(agg_sh.at[pl.ds(r0, RPT)],
                    agg_out.at[cid, pl.ds(r0, RPT)])


BM = 2000


def _tc_layer_kernel(agg_ref, h_ref, deg_ref, w_ref, b_ref, o_ref):
    d = deg_ref[0, :, 0:1] + deg_ref[1, :, 0:1]
    z = (agg_ref[0] + agg_ref[1] + h_ref[...]) / (d + 1.0)
    o_ref[...] = jnp.maximum(
        jnp.dot(z, w_ref[...], preferred_element_type=jnp.float32)
        + b_ref[...], 0.0)


def _tc_layer(agg, h, deg, W, b):
    return pl.pallas_call(
        _tc_layer_kernel,
        grid=(N_NODES // BM,),
        in_specs=[
            pl.BlockSpec((2, BM, D), lambda i: (0, i, 0)),
            pl.BlockSpec((BM, D), lambda i: (i, 0)),
            pl.BlockSpec((2, BM, D), lambda i: (0, i, 0)),
            pl.BlockSpec((D, D), lambda i: (0, 0)),
            pl.BlockSpec((1, D), lambda i: (0, 0)),
        ],
        out_specs=pl.BlockSpec((BM, D), lambda i: (i, 0)),
        out_shape=jax.ShapeDtypeStruct((N_NODES, D), jnp.float32),
    )(agg, h, deg, W, b.reshape(1, D))


def _tc_final_kernel(agg_ref, h_ref, deg_ref, w3_ref, b3_ref, wv_ref,
                     bv_ref, wc_ref, bc_ref, o_ref, acc):
    i = pl.program_id(0)
    d = deg_ref[0, :, 0:1] + deg_ref[1, :, 0:1]
    z = (agg_ref[0] + agg_ref[1] + h_ref[...]) / (d + 1.0)
    h3 = jnp.dot(z, w3_ref[...], preferred_element_type=jnp.float32) \
        + b3_ref[...]
    wv = jax.nn.sigmoid(
        jnp.dot(h3, wv_ref[...], preferred_element_type=jnp.float32)
        + bv_ref[...])
    part = jnp.sum(wv * h3, axis=0, keepdims=True)

    @pl.when(i == 0)
    def _():
        acc[...] = jnp.zeros_like(acc)

    acc[...] += part

    @pl.when(i == pl.num_programs(0) - 1)
    def _():
        hg = acc[...] / float(N_NODES)
        o_ref[...] = jnp.dot(hg, wc_ref[...],
                             preferred_element_type=jnp.float32) + bc_ref[...]


def _tc_final(agg, h, deg, W3, b3, Wv, bv, Wc, bc):
    return pl.pallas_call(
        _tc_final_kernel,
        grid=(N_NODES // BM,),
        in_specs=[
            pl.BlockSpec((2, BM, D), lambda i: (0, i, 0)),
            pl.BlockSpec((BM, D), lambda i: (i, 0)),
            pl.BlockSpec((2, BM, D), lambda i: (0, i, 0)),
            pl.BlockSpec((D, D), lambda i: (0, 0)),
            pl.BlockSpec((1, D), lambda i: (0, 0)),
            pl.BlockSpec((D, 1), lambda i: (0, 0)),
            pl.BlockSpec((1, 1), lambda i: (0, 0)),
            pl.BlockSpec((D, D_OUT), lambda i: (0, 0)),
            pl.BlockSpec((1, D_OUT), lambda i: (0, 0)),
        ],
        out_specs=pl.BlockSpec((1, D_OUT), lambda i: (0, 0)),
        out_shape=jax.ShapeDtypeStruct((1, D_OUT), jnp.float32),
        scratch_shapes=[pltpu.VMEM((1, D), jnp.float32)],
    )(agg, h, deg, W3, b3.reshape(1, D), Wv, bv.reshape(1, 1),
      Wc, bc.reshape(1, D_OUT))


@jax.jit
def _run(x, edge_index, W1, b1, W2, b2, W3, b3, Wv, bv, Wc, bc):
    npad = NW * CPW * EC - N_EDGES
    src = jnp.concatenate(
        [edge_index[0].astype(jnp.int32), jnp.zeros((npad,), jnp.int32)])
    dst = jnp.concatenate(
        [edge_index[1].astype(jnp.int32),
         N_NODES + (jnp.arange(npad, dtype=jnp.int32) % (NP - N_NODES))])
    src = src.reshape(NW * CPW, EC)
    dst = dst.reshape(NW * CPW, EC)

    deg = _sc_deg(dst)
    agg1 = _sc_agg(x, src, dst)
    h1 = _tc_layer(agg1, x, deg, W1, b1)
    agg2 = _sc_agg(h1, src, dst)
    h2 = _tc_layer(agg2, h1, deg, W2, b2)
    agg3 = _sc_agg(h2, src, dst)
    return _tc_final(agg3, h2, deg, W3, b3, Wv, bv, Wc, bc)


def kernel(x, edge_index, W1, b1, W2, b2, W3, b3, Wv, bv, Wc, bc):
    return _run(x, edge_index, W1, b1, W2, b2, W3, b3, Wv, bv, Wc, bc)
